# baseline (speedup 1.0000x reference)
import functools

import jax
import jax.numpy as jnp
from jax import lax
from jax.experimental import pallas as pl
from jax.experimental.pallas import tpu as pltpu
from jax.experimental.pallas import tpu_sc as plsc

N = 10000
E = 320000
D = 128
R2 = 32
NPAD = 10240
BLK = 512

NC = 2
NS = 16
RNG = 320
HRNG = 160
CHUNK = 3200
NCH = E // CHUNK
GB = 64
CPAD = CHUNK + GB


def _sc_agg_body(nf_hbm, src_hbm, dst_hbm, attr_hbm, rel_hbm,
                 sums, sqs, mxs, mns,
                 rel_v, dstcA, srccA, attrcA, dstcB, srccB, attrcB,
                 cdst, csrc, cattr,
                 gidx, rows, accsum, accsq, accmx, accmn,
                 gsem, semA, semB):
    c = lax.axis_index("c")
    s = lax.axis_index("s")
    o = c * NS + s
    lo = o * RNG
    iota = lax.iota(jnp.int32, NS)

    pltpu.sync_copy(rel_hbm, rel_v)

    def init_accs():
        def irow(r, _):
            for j in range(8):
                sl = pl.ds(j * 16, 16)
                accsum[r, sl] = jnp.zeros((16,), jnp.float32)
                accsq[r, sl] = jnp.zeros((16,), jnp.float32)
                accmx[r, sl] = jnp.full((16,), -jnp.inf, jnp.float32)
                accmn[r, sl] = jnp.full((16,), jnp.inf, jnp.float32)
            return 0
        lax.fori_loop(0, HRNG + 1, irow, 0)

    bufsA = (dstcA, srccA, attrcA)
    bufsB = (dstcB, srccB, attrcB)
    hbms = (dst_hbm, src_hbm, attr_hbm)

    def cmap(ch):
        ch2 = ch + o * 3
        return jnp.where(ch2 >= NCH, ch2 - NCH, ch2)

    def start_loads(ch, bufs, sem):
        c0 = cmap(ch) * CHUNK
        for hb, bf in zip(hbms, bufs):
            pltpu.async_copy(hb.at[pl.ds(c0, CHUNK)], bf, sem)

    def wait_loads(ch, bufs, sem):
        c0 = cmap(ch) * CHUNK
        for hb, bf in zip(hbms, bufs):
            pltpu.make_async_copy(hb.at[pl.ds(c0, CHUNK)], bf, sem).wait()

    def chunk_body(ch, bufs):
        dstc, srcc, attrc = bufs
        cbase = lo + h * HRNG

        def fbody(t, cntv):
            offl = [t * 80 + u * 16 for u in range(5)]
            d2s = [dstc[pl.ds(off, 16)] - cbase for off in offl]
            mks = [plsc.bitcast(d2, jnp.uint32) < jnp.uint32(HRNG)
                   for d2 in d2s]
            css = [plsc.cumsum(jnp.where(m, 1, 0)) for m in mks]
            pcs = [plsc.all_reduce_population_count(m) for m in mks]
            base = cntv
            for u in range(5):
                offs = base + css[u] - 1
                plsc.store_scatter(cdst, [offs], d2s[u], mask=mks[u])
                plsc.store_scatter(csrc, [offs], srcc[pl.ds(offl[u], 16)],
                                   mask=mks[u])
                plsc.store_scatter(cattr, [offs], attrc[pl.ds(offl[u], 16)],
                                   mask=mks[u])
                base = base + pcs[u]
            return base
        cntv = lax.fori_loop(0, CHUNK // 80, fbody, jnp.zeros((16,), jnp.int32))
        kcnt = cntv[0]
        kpad = (kcnt + (GB - 1)) & (-GB)
        for u in range(GB // 16):
            pidx = kcnt + u * 16 + iota
            pm = pidx < kpad
            plsc.store_scatter(cdst, [pidx],
                               jnp.full((16,), HRNG, jnp.int32), mask=pm)
            plsc.store_scatter(csrc, [pidx], o * 16 + iota, mask=pm)
            plsc.store_scatter(cattr, [pidx], jnp.zeros((16,), jnp.int32),
                               mask=pm)

        def bbody(b, _):
            g0 = b * GB
            for u in range(GB // 16):
                gidx[pl.ds(u * 16, 16)] = csrc[pl.ds(g0 + u * 16, 16)]
            pltpu.async_copy(nf_hbm.at[gidx], rows, gsem).wait()

            cols = [j * 16 + iota for j in range(8)]

            def ibody(i4, _):
                tlocs, msq = [], []
                for q in range(4):
                    i = 4 * i4 + q
                    iiv = jnp.full((16,), g0 + i, jnp.int32)
                    attr_b = plsc.load_gather(cattr, [iiv])
                    tlocs.append(plsc.load_gather(cdst, [iiv]))
                    srows = [rows[i, pl.ds(j * 16, 16)] for j in range(8)]
                    rrows = [plsc.load_gather(rel_v, [attr_b, cols[j]])
                             for j in range(8)]
                    msq.append([srows[j] * rrows[j] for j in range(8)])
                for q in range(4):
                    tloc, ms = tlocs[q], msq[q]
                    curx = [plsc.load_gather(accmx, [tloc, cols[j]])
                            for j in range(8)]
                    curn = [plsc.load_gather(accmn, [tloc, cols[j]])
                            for j in range(8)]
                    for j in range(8):
                        plsc.addupdate_scatter(accsum, [tloc, cols[j]], ms[j])
                        plsc.addupdate_scatter(accsq, [tloc, cols[j]],
                                               ms[j] * ms[j])
                        plsc.store_scatter(accmx, [tloc, cols[j]],
                                           jnp.maximum(curx[j], ms[j]))
                        plsc.store_scatter(accmn, [tloc, cols[j]],
                                           jnp.minimum(curn[j], ms[j]))
                return 0
            lax.fori_loop(0, GB // 4, ibody, 0)
            return 0
        lax.fori_loop(0, kpad >> 6, bbody, 0)

    for h in range(2):
        init_accs()
        start_loads(0, bufsA, semA)

        def pair_body(t, _, h=h):
            ch = 2 * t
            start_loads(ch + 1, bufsB, semB)
            wait_loads(ch, bufsA, semA)
            chunk_body(ch, bufsA)

            @pl.when(t < NCH // 2 - 1)
            def _():
                start_loads(ch + 2, bufsA, semA)
            wait_loads(ch + 1, bufsB, semB)
            chunk_body(ch + 1, bufsB)
            return 0
        lax.fori_loop(0, NCH // 2, pair_body, 0)

        hb = lo + h * HRNG
        pltpu.sync_copy(accsum.at[pl.ds(0, HRNG)], sums.at[pl.ds(hb, HRNG)])
        pltpu.sync_copy(accsq.at[pl.ds(0, HRNG)], sqs.at[pl.ds(hb, HRNG)])
        pltpu.sync_copy(accmx.at[pl.ds(0, HRNG)], mxs.at[pl.ds(hb, HRNG)])
        pltpu.sync_copy(accmn.at[pl.ds(0, HRNG)], mns.at[pl.ds(hb, HRNG)])


@functools.partial(
    pl.kernel,
    out_type=[jax.ShapeDtypeStruct((NPAD, D), jnp.float32) for _ in range(4)],
    mesh=plsc.VectorSubcoreMesh(core_axis_name="c", subcore_axis_name="s"),
    compiler_params=pltpu.CompilerParams(needs_layout_passes=False),
    scratch_types=[
        pltpu.VMEM((R2, D), jnp.float32),
        pltpu.VMEM((CHUNK,), jnp.int32),
        pltpu.VMEM((CHUNK,), jnp.int32),
        pltpu.VMEM((CHUNK,), jnp.int32),
        pltpu.VMEM((CHUNK,), jnp.int32),
        pltpu.VMEM((CHUNK,), jnp.int32),
        pltpu.VMEM((CHUNK,), jnp.int32),
        pltpu.VMEM((CPAD,), jnp.int32),
        pltpu.VMEM((CPAD,), jnp.int32),
        pltpu.VMEM((CPAD,), jnp.int32),
        pltpu.VMEM((GB,), jnp.int32),
        pltpu.VMEM((GB, D), jnp.float32),
        pltpu.VMEM((HRNG + 1, D), jnp.float32),
        pltpu.VMEM((HRNG + 1, D), jnp.float32),
        pltpu.VMEM((HRNG + 1, D), jnp.float32),
        pltpu.VMEM((HRNG + 1, D), jnp.float32),
        pltpu.SemaphoreType.DMA,
        pltpu.SemaphoreType.DMA,
        pltpu.SemaphoreType.DMA,
    ],
)
def _sc_agg(nf_hbm, src_hbm, dst_hbm, attr_hbm, rel_hbm,
            sums, sqs, mxs, mns, *scratch):
    _sc_agg_body(nf_hbm, src_hbm, dst_hbm, attr_hbm, rel_hbm,
                 sums, sqs, mxs, mns, *scratch)


def _prep_body(qT_ref, WT_ref, brel_ref, degpad_ref, rel_ref, smean_ref):
    rel_ref[...] = (
        jnp.dot(qT_ref[...], WT_ref[...], preferred_element_type=jnp.float32)
        + brel_ref[...]
    )
    smean_ref[...] = jnp.sum(jnp.log(degpad_ref[...] + 1.0)).reshape(1, 1) / N


def _prep(qT, WT, brel, degpad):
    return pl.pallas_call(
        _prep_body,
        out_shape=(
            jax.ShapeDtypeStruct((1, R2 * D), jnp.float32),
            jax.ShapeDtypeStruct((1, 1), jnp.float32),
        ),
    )(qT, WT, brel, degpad)


def _final_body(nf_ref, sum_ref, sq_ref, mx_ref, mn_ref, bnd_ref, deg_ref,
                WgT_ref, blin_ref, smean_ref, out_ref):
    deg = deg_ref[...] + 1.0
    bnd = bnd_ref[...]
    mean = (sum_ref[...] + bnd) / deg
    sq_mean = (sq_ref[...] + bnd * bnd) / deg
    mx = mx_ref[...]
    mn = mn_ref[...]
    mx = jnp.maximum(jnp.where(jnp.isfinite(mx), mx, 0.0), bnd)
    mn = jnp.minimum(jnp.where(jnp.isfinite(mn), mn, 0.0), bnd)
    std = jnp.sqrt(jnp.clip(sq_mean - mean * mean, 1e-06, None))
    scale = jnp.log(deg)
    s1 = scale / (smean_ref[0, 0] + 1e-10)
    s2 = 1.0 / jnp.clip(s1, 0.01, None)
    X = jnp.concatenate(
        [nf_ref[...],
         mean, mean * s1, mean * s2,
         mx, mx * s1, mx * s2,
         mn, mn * s1, mn * s2,
         std, std * s1, std * s2], axis=-1)
    acc = jnp.dot(X, WgT_ref[...], preferred_element_type=jnp.float32)
    out_ref[...] = jnp.maximum(acc + blin_ref[...], 0.0)


def _final(nf, sum_, sq, mx, mn, bnd, deg, WgT, blin, smean):
    grid = NPAD // BLK
    row = pl.BlockSpec((BLK, D), lambda i: (i, 0))
    const2 = lambda shape: pl.BlockSpec(shape, lambda i: (0, 0))
    return pl.pallas_call(
        _final_body,
        grid=(grid,),
        in_specs=[row, row, row, row, row, row, row,
                  const2((13 * D, D)), const2((1, D)), const2((1, 1))],
        out_specs=row,
        out_shape=jax.ShapeDtypeStruct((NPAD, D), jnp.float32),
    )(nf, sum_, sq, mx, mn, bnd, deg, WgT, blin, smean)


def kernel(node_features, query, boundary, degree_out, edge_index, edge_attr,
           W_rel, b_rel, W_lin, b_lin):
    src = edge_index[0].astype(jnp.int32)
    dst = edge_index[1].astype(jnp.int32)
    attr = edge_attr.astype(jnp.int32)

    qT = query.reshape(1, D)
    WT = W_rel.T
    brel = b_rel.reshape(1, R2 * D)
    degpad = jnp.pad(degree_out, (0, NPAD - N)).reshape(NPAD // D, D)
    W_upd = W_lin[:, D:].reshape(D, D, 4, 3).transpose(0, 2, 3, 1).reshape(D, 12 * D)
    WgT = jnp.concatenate([W_lin[:, :D], W_upd], axis=1).T
    blin = b_lin.reshape(1, D)

    rel_flat, smean = _prep(qT, WT, brel, degpad)
    rel = rel_flat.reshape(R2, D)

    sum_agg, sq_agg, mx_agg, mn_agg = _sc_agg(node_features, src, dst, attr, rel)

    nf_p = jnp.pad(node_features, ((0, NPAD - N), (0, 0)))
    bnd_p = jnp.pad(boundary, ((0, NPAD - N), (0, 0)))
    deg_b = jnp.broadcast_to(jnp.pad(degree_out, (0, NPAD - N))[:, None], (NPAD, D))

    out = _final(nf_p, sum_agg, sq_agg, mx_agg, mn_agg, bnd_p, deg_b,
                 WgT, blin, smean)
    return out[:N]


# speedup vs baseline: 1.0195x; 1.0195x over previous
import functools

import jax
import jax.numpy as jnp
from jax import lax
from jax.experimental import pallas as pl
from jax.experimental.pallas import tpu as pltpu
from jax.experimental.pallas import tpu_sc as plsc

N = 10000
E = 320000
D = 128
R2 = 32
NPAD = 10240
BLK = 512

NC = 2
NS = 16
RNG = 320
HRNG = 160
CHUNK = 3200
NCH = E // CHUNK
GB = 64
CPAD = CHUNK + GB


def _sc_agg_body(nf_hbm, src_hbm, dst_hbm, attr_hbm, rel_hbm,
                 sums, sqs, mxs, mns,
                 rel_v, dstcA, srccA, attrcA, dstcB, srccB, attrcB,
                 cdst, csrc, cattr,
                 gidx, rows, accsum, accsq, accmx, accmn,
                 gsem, semA, semB):
    c = lax.axis_index("c")
    s = lax.axis_index("s")
    o = c * NS + s
    lo = o * RNG
    iota = lax.iota(jnp.int32, NS)

    pltpu.sync_copy(rel_hbm, rel_v)

    def init_accs():
        def irow(r, _):
            for j in range(8):
                sl = pl.ds(j * 16, 16)
                accsum[r, sl] = jnp.zeros((16,), jnp.float32)
                accsq[r, sl] = jnp.zeros((16,), jnp.float32)
                accmx[r, sl] = jnp.full((16,), -jnp.inf, jnp.float32)
                accmn[r, sl] = jnp.full((16,), jnp.inf, jnp.float32)
            return 0
        lax.fori_loop(0, HRNG + 1, irow, 0)

    bufsA = (dstcA, srccA, attrcA)
    bufsB = (dstcB, srccB, attrcB)
    hbms = (dst_hbm, src_hbm, attr_hbm)

    def cmap(ch):
        ch2 = ch + o * 3
        return jnp.where(ch2 >= NCH, ch2 - NCH, ch2)

    def start_loads(ch, bufs, sem):
        c0 = cmap(ch) * CHUNK
        for hb, bf in zip(hbms, bufs):
            pltpu.async_copy(hb.at[pl.ds(c0, CHUNK)], bf, sem)

    def wait_loads(ch, bufs, sem):
        c0 = cmap(ch) * CHUNK
        for hb, bf in zip(hbms, bufs):
            pltpu.make_async_copy(hb.at[pl.ds(c0, CHUNK)], bf, sem).wait()

    def chunk_body(ch, bufs):
        dstc, srcc, attrc = bufs
        cbase = lo + h * HRNG

        def fbody(t, cntv):
            offl = [t * 80 + u * 16 for u in range(5)]
            d2s = [dstc[pl.ds(off, 16)] - cbase for off in offl]
            mks = [plsc.bitcast(d2, jnp.uint32) < jnp.uint32(HRNG)
                   for d2 in d2s]
            css = [plsc.cumsum(jnp.where(m, 1, 0)) for m in mks]
            pcs = [plsc.all_reduce_population_count(m) for m in mks]
            base = cntv
            for u in range(5):
                offs = base + css[u] - 1
                plsc.store_scatter(cdst, [offs], d2s[u], mask=mks[u])
                plsc.store_scatter(csrc, [offs], srcc[pl.ds(offl[u], 16)],
                                   mask=mks[u])
                plsc.store_scatter(cattr, [offs], attrc[pl.ds(offl[u], 16)],
                                   mask=mks[u])
                base = base + pcs[u]
            return base
        cntv = lax.fori_loop(0, CHUNK // 80, fbody, jnp.zeros((16,), jnp.int32))
        kcnt = cntv[0]
        kpad = (kcnt + (GB - 1)) & (-GB)
        for u in range(GB // 16):
            pidx = kcnt + u * 16 + iota
            pm = pidx < kpad
            plsc.store_scatter(cdst, [pidx],
                               jnp.full((16,), HRNG, jnp.int32), mask=pm)
            plsc.store_scatter(csrc, [pidx], o * 16 + iota, mask=pm)
            plsc.store_scatter(cattr, [pidx], jnp.zeros((16,), jnp.int32),
                               mask=pm)

        def bbody(b, _):
            g0 = b * GB
            for u in range(GB // 16):
                gidx[pl.ds(u * 16, 16)] = csrc[pl.ds(g0 + u * 16, 16)]
            pltpu.async_copy(nf_hbm.at[gidx], rows, gsem).wait()

            def ibody(i2, _):
                for q in range(2):
                    i = 2 * i2 + q
                    iiv = jnp.full((16,), g0 + i, jnp.int32)
                    attr_b = plsc.load_gather(cattr, [iiv])
                    tloc = plsc.load_gather(cdst, [iiv])
                    cols = [j * 16 + iota for j in range(8)]
                    srows = [rows[i, pl.ds(j * 16, 16)] for j in range(8)]
                    rrows = [plsc.load_gather(rel_v, [attr_b, cols[j]])
                             for j in range(8)]
                    ms = [srows[j] * rrows[j] for j in range(8)]
                    curx = [plsc.load_gather(accmx, [tloc, cols[j]])
                            for j in range(8)]
                    curn = [plsc.load_gather(accmn, [tloc, cols[j]])
                            for j in range(8)]
                    for j in range(8):
                        plsc.addupdate_scatter(accsum, [tloc, cols[j]], ms[j])
                        plsc.addupdate_scatter(accsq, [tloc, cols[j]],
                                               ms[j] * ms[j])
                        plsc.store_scatter(accmx, [tloc, cols[j]],
                                           jnp.maximum(curx[j], ms[j]))
                        plsc.store_scatter(accmn, [tloc, cols[j]],
                                           jnp.minimum(curn[j], ms[j]))
                return 0
            lax.fori_loop(0, GB // 2, ibody, 0)
            return 0
        lax.fori_loop(0, kpad >> 6, bbody, 0)

    for h in range(2):
        init_accs()
        start_loads(0, bufsA, semA)

        def pair_body(t, _, h=h):
            ch = 2 * t
            start_loads(ch + 1, bufsB, semB)
            wait_loads(ch, bufsA, semA)
            chunk_body(ch, bufsA)

            @pl.when(t < NCH // 2 - 1)
            def _():
                start_loads(ch + 2, bufsA, semA)
            wait_loads(ch + 1, bufsB, semB)
            chunk_body(ch + 1, bufsB)
            return 0
        lax.fori_loop(0, NCH // 2, pair_body, 0)

        hb = lo + h * HRNG
        pltpu.sync_copy(accsum.at[pl.ds(0, HRNG)], sums.at[pl.ds(hb, HRNG)])
        pltpu.sync_copy(accsq.at[pl.ds(0, HRNG)], sqs.at[pl.ds(hb, HRNG)])
        pltpu.sync_copy(accmx.at[pl.ds(0, HRNG)], mxs.at[pl.ds(hb, HRNG)])
        pltpu.sync_copy(accmn.at[pl.ds(0, HRNG)], mns.at[pl.ds(hb, HRNG)])


@functools.partial(
    pl.kernel,
    out_type=[jax.ShapeDtypeStruct((NPAD, D), jnp.float32) for _ in range(4)],
    mesh=plsc.VectorSubcoreMesh(core_axis_name="c", subcore_axis_name="s"),
    compiler_params=pltpu.CompilerParams(needs_layout_passes=False),
    scratch_types=[
        pltpu.VMEM((R2, D), jnp.float32),
        pltpu.VMEM((CHUNK,), jnp.int32),
        pltpu.VMEM((CHUNK,), jnp.int32),
        pltpu.VMEM((CHUNK,), jnp.int32),
        pltpu.VMEM((CHUNK,), jnp.int32),
        pltpu.VMEM((CHUNK,), jnp.int32),
        pltpu.VMEM((CHUNK,), jnp.int32),
        pltpu.VMEM((CPAD,), jnp.int32),
        pltpu.VMEM((CPAD,), jnp.int32),
        pltpu.VMEM((CPAD,), jnp.int32),
        pltpu.VMEM((GB,), jnp.int32),
        pltpu.VMEM((GB, D), jnp.float32),
        pltpu.VMEM((HRNG + 1, D), jnp.float32),
        pltpu.VMEM((HRNG + 1, D), jnp.float32),
        pltpu.VMEM((HRNG + 1, D), jnp.float32),
        pltpu.VMEM((HRNG + 1, D), jnp.float32),
        pltpu.SemaphoreType.DMA,
        pltpu.SemaphoreType.DMA,
        pltpu.SemaphoreType.DMA,
    ],
)
def _sc_agg(nf_hbm, src_hbm, dst_hbm, attr_hbm, rel_hbm,
            sums, sqs, mxs, mns, *scratch):
    _sc_agg_body(nf_hbm, src_hbm, dst_hbm, attr_hbm, rel_hbm,
                 sums, sqs, mxs, mns, *scratch)


def _prep_body(qT_ref, WT_ref, brel_ref, degpad_ref, rel_ref, smean_ref):
    rel_ref[...] = (
        jnp.dot(qT_ref[...], WT_ref[...], preferred_element_type=jnp.float32)
        + brel_ref[...]
    )
    smean_ref[...] = jnp.sum(jnp.log(degpad_ref[...] + 1.0)).reshape(1, 1) / N


def _prep(qT, WT, brel, degpad):
    return pl.pallas_call(
        _prep_body,
        out_shape=(
            jax.ShapeDtypeStruct((1, R2 * D), jnp.float32),
            jax.ShapeDtypeStruct((1, 1), jnp.float32),
        ),
    )(qT, WT, brel, degpad)


def _final_body(nf_ref, sum_ref, sq_ref, mx_ref, mn_ref, bnd_ref, deg_ref,
                WgT_ref, blin_ref, smean_ref, out_ref):
    deg = deg_ref[...] + 1.0
    bnd = bnd_ref[...]
    mean = (sum_ref[...] + bnd) / deg
    sq_mean = (sq_ref[...] + bnd * bnd) / deg
    mx = mx_ref[...]
    mn = mn_ref[...]
    mx = jnp.maximum(jnp.where(jnp.isfinite(mx), mx, 0.0), bnd)
    mn = jnp.minimum(jnp.where(jnp.isfinite(mn), mn, 0.0), bnd)
    std = jnp.sqrt(jnp.clip(sq_mean - mean * mean, 1e-06, None))
    scale = jnp.log(deg)
    s1 = scale / (smean_ref[0, 0] + 1e-10)
    s2 = 1.0 / jnp.clip(s1, 0.01, None)
    X = jnp.concatenate(
        [nf_ref[...],
         mean, mean * s1, mean * s2,
         mx, mx * s1, mx * s2,
         mn, mn * s1, mn * s2,
         std, std * s1, std * s2], axis=-1)
    acc = jnp.dot(X, WgT_ref[...], preferred_element_type=jnp.float32)
    out_ref[...] = jnp.maximum(acc + blin_ref[...], 0.0)


def _final(nf, sum_, sq, mx, mn, bnd, deg, WgT, blin, smean):
    grid = NPAD // BLK
    row = pl.BlockSpec((BLK, D), lambda i: (i, 0))
    const2 = lambda shape: pl.BlockSpec(shape, lambda i: (0, 0))
    return pl.pallas_call(
        _final_body,
        grid=(grid,),
        in_specs=[row, row, row, row, row, row, row,
                  const2((13 * D, D)), const2((1, D)), const2((1, 1))],
        out_specs=row,
        out_shape=jax.ShapeDtypeStruct((NPAD, D), jnp.float32),
    )(nf, sum_, sq, mx, mn, bnd, deg, WgT, blin, smean)


def kernel(node_features, query, boundary, degree_out, edge_index, edge_attr,
           W_rel, b_rel, W_lin, b_lin):
    src = edge_index[0].astype(jnp.int32)
    dst = edge_index[1].astype(jnp.int32)
    attr = edge_attr.astype(jnp.int32)

    qT = query.reshape(1, D)
    WT = W_rel.T
    brel = b_rel.reshape(1, R2 * D)
    degpad = jnp.pad(degree_out, (0, NPAD - N)).reshape(NPAD // D, D)
    W_upd = W_lin[:, D:].reshape(D, D, 4, 3).transpose(0, 2, 3, 1).reshape(D, 12 * D)
    WgT = jnp.concatenate([W_lin[:, :D], W_upd], axis=1).T
    blin = b_lin.reshape(1, D)

    rel_flat, smean = _prep(qT, WT, brel, degpad)
    rel = rel_flat.reshape(R2, D)

    sum_agg, sq_agg, mx_agg, mn_agg = _sc_agg(node_features, src, dst, attr, rel)

    nf_p = jnp.pad(node_features, ((0, NPAD - N), (0, 0)))
    bnd_p = jnp.pad(boundary, ((0, NPAD - N), (0, 0)))
    deg_b = jnp.broadcast_to(jnp.pad(degree_out, (0, NPAD - N))[:, None], (NPAD, D))

    out = _final(nf_p, sum_agg, sq_agg, mx_agg, mn_agg, bnd_p, deg_b,
                 WgT, blin, smean)
    return out[:N]
